# baseline (speedup 1.0000x reference)
import functools

import jax
import jax.numpy as jnp
from jax import lax
from jax.experimental import pallas as pl
from jax.experimental.pallas import tpu as pltpu
from jax.experimental.pallas import tpu_sc as plsc

N_CLASS = 1000000
EMBED_DIM = 32
BATCH = 16384
FIELDS = 26
B_FLAT = BATCH * FIELDS

_NC = 2
_NS = 16
_NW = _NC * _NS

_CHUNK = 256
_NCHUNKS = B_FLAT // (_NW * _CHUNK)
_IDX_ROWS = B_FLAT // _CHUNK


def _gather_body(idx_hbm, table_hbm, out_hbm, idx_v, rows_v0, rows_v1,
                 sem_g0, sem_g1, sem_w0, sem_w1):
    c = lax.axis_index("c")
    s = lax.axis_index("s")
    wid = s * _NC + c
    r0 = wid * _NCHUNKS

    pltpu.sync_copy(idx_hbm.at[pl.ds(r0, _NCHUNKS)], idx_v)

    rows_v = (rows_v0, rows_v1)
    sem_g = (sem_g0, sem_g1)
    sem_w = (sem_w0, sem_w1)
    gathers = [None] * _NCHUNKS
    writes = [None] * _NCHUNKS

    def start_chunk(i):
        p = i % 2
        gathers[i] = pltpu.async_copy(
            table_hbm.at[idx_v.at[i]], rows_v[p], sem_g[p])

    def drain_chunk(i):
        p = i % 2
        pos0 = (r0 + i) * _CHUNK
        f = pos0 // BATCH
        b0 = pos0 - f * BATCH
        gathers[i].wait()
        writes[i] = pltpu.async_copy(
            rows_v[p].at[pl.ds(0, _CHUNK), pl.ds(0, EMBED_DIM)],
            out_hbm.at[pl.ds(b0, _CHUNK), f, pl.ds(0, EMBED_DIM)],
            sem_w[p])

    for i in range(_NCHUNKS):
        if i >= 2:
            writes[i - 2].wait()
        start_chunk(i)
        if i >= 1:
            drain_chunk(i - 1)
    drain_chunk(_NCHUNKS - 1)
    writes[_NCHUNKS - 2].wait()
    writes[_NCHUNKS - 1].wait()


@jax.jit
def _gather(idx2d, table):
    mesh = plsc.VectorSubcoreMesh(core_axis_name="c", subcore_axis_name="s")
    kern = functools.partial(
        pl.kernel,
        mesh=mesh,
        out_type=jax.ShapeDtypeStruct((BATCH, 32, 128), jnp.float32),
        scratch_types=[
            pltpu.VMEM((_NCHUNKS, _CHUNK), jnp.int32),
            pltpu.VMEM((_CHUNK, 128), jnp.float32),
            pltpu.VMEM((_CHUNK, 128), jnp.float32),
            pltpu.SemaphoreType.DMA,
            pltpu.SemaphoreType.DMA,
            pltpu.SemaphoreType.DMA,
            pltpu.SemaphoreType.DMA,
        ],
        compiler_params=pltpu.CompilerParams(use_tc_tiling_on_sc=False),
    )(_gather_body)
    return kern(idx2d, table)


def kernel(z_category, categ_embed_weight):
    idx2d = z_category.astype(jnp.int32).T.reshape(_IDX_ROWS, _CHUNK)
    wt128 = jnp.pad(categ_embed_weight, ((0, 0), (0, 128 - EMBED_DIM)))
    out_pad = _gather(idx2d, wt128)
    return out_pad[:, :FIELDS, :EMBED_DIM]


# speedup vs baseline: 1.0819x; 1.0819x over previous
import functools

import jax
import jax.numpy as jnp
from jax import lax
from jax.experimental import pallas as pl
from jax.experimental.pallas import tpu as pltpu
from jax.experimental.pallas import tpu_sc as plsc

N_CLASS = 1000000
EMBED_DIM = 32
BATCH = 16384
FIELDS = 26
B_FLAT = BATCH * FIELDS

_NC = 2
_NS = 16
_NW = _NC * _NS

_CHUNK = 1024
_NCHUNKS = B_FLAT // (_NW * _CHUNK)
_IDX_ROWS = B_FLAT // _CHUNK


def _gather_body(idx_hbm, table_hbm, out_hbm, idx_v, rows_v0, rows_v1,
                 sem_g0, sem_g1, sem_w0, sem_w1):
    c = lax.axis_index("c")
    s = lax.axis_index("s")
    wid = s * _NC + c
    r0 = wid * _NCHUNKS

    pltpu.sync_copy(idx_hbm.at[pl.ds(r0, _NCHUNKS)], idx_v)

    rows_v = (rows_v0, rows_v1)
    sem_g = (sem_g0, sem_g1)
    sem_w = (sem_w0, sem_w1)
    gathers = [None] * _NCHUNKS
    writes = [None] * _NCHUNKS

    def start_chunk(i):
        p = i % 2
        gathers[i] = pltpu.async_copy(
            table_hbm.at[idx_v.at[i]], rows_v[p], sem_g[p])

    def drain_chunk(i):
        p = i % 2
        pos0 = (r0 + i) * _CHUNK
        f = pos0 // BATCH
        b0 = pos0 - f * BATCH
        gathers[i].wait()
        writes[i] = pltpu.async_copy(
            rows_v[p],
            out_hbm.at[pl.ds(b0, _CHUNK), f, pl.ds(0, EMBED_DIM)],
            sem_w[p])

    for i in range(_NCHUNKS):
        if i >= 2:
            writes[i - 2].wait()
        start_chunk(i)
        if i >= 1:
            drain_chunk(i - 1)
    drain_chunk(_NCHUNKS - 1)
    writes[_NCHUNKS - 2].wait()
    writes[_NCHUNKS - 1].wait()


@jax.jit
def _gather(idx2d, table):
    mesh = plsc.VectorSubcoreMesh(core_axis_name="c", subcore_axis_name="s")
    kern = functools.partial(
        pl.kernel,
        mesh=mesh,
        out_type=jax.ShapeDtypeStruct((BATCH, 32, 128), jnp.float32),
        scratch_types=[
            pltpu.VMEM((_NCHUNKS, _CHUNK), jnp.int32),
            pltpu.VMEM((_CHUNK, EMBED_DIM), jnp.float32),
            pltpu.VMEM((_CHUNK, EMBED_DIM), jnp.float32),
            pltpu.SemaphoreType.DMA,
            pltpu.SemaphoreType.DMA,
            pltpu.SemaphoreType.DMA,
            pltpu.SemaphoreType.DMA,
        ],
        compiler_params=pltpu.CompilerParams(use_tc_tiling_on_sc=False),
    )(_gather_body)
    return kern(idx2d, table)


def kernel(z_category, categ_embed_weight):
    idx2d = z_category.astype(jnp.int32).T.reshape(_IDX_ROWS, _CHUNK)
    out_pad = _gather(idx2d, categ_embed_weight)
    return out_pad[:, :FIELDS, :EMBED_DIM]
